# baseline (speedup 1.0000x reference)
import functools
import math

import jax
import jax.numpy as jnp
from jax import lax
from jax.experimental import pallas as pl
from jax.experimental.pallas import tpu as pltpu
from jax.experimental.pallas import tpu_sc as plsc

NUM_BUCKETS = 32
NUM_HEADS = 16
MAX_DISTANCE = 128
S_Q = 2048
S_K = 2048
D2_LEN = 4224
SLAB_W = 3968
WIN_LEN = 3984
HEADS_PER_CORE = NUM_HEADS // 2


def _bucket_values(d, table_ref, h):
    nb = NUM_BUCKETS // 2
    base = jnp.where(d > 0, nb, 0).astype(jnp.int32)
    rp = jnp.abs(d)
    max_exact = nb // 2
    is_small = rp < max_exact
    rpf = rp.astype(jnp.float32)
    large = max_exact + (
        jnp.log(rpf / max_exact)
        / math.log(MAX_DISTANCE / max_exact)
        * (nb - max_exact)
    ).astype(jnp.int32)
    large = jnp.minimum(large, nb - 1)
    bucket = base + jnp.where(is_small, rp, large)
    val = jnp.zeros(d.shape, jnp.float32)
    for b in range(NUM_BUCKETS):
        val = jnp.where(bucket == b, table_ref[b, h], val)
    return val


def _diag_kernel(off_ref, table_ref, out_ref):
    h = pl.program_id(0)
    i = jax.lax.broadcasted_iota(jnp.int32, (8, D2_LEN), 0)
    j = jax.lax.broadcasted_iota(jnp.int32, (8, D2_LEN), 1)
    d = j + 7 - i - 2047 + off_ref[0]
    out_ref[0] = _bucket_values(d, table_ref, h)


def _sc_expand(diag_hbm, out_hbm,
               win_v, rows0, rows1, slab0, slab1, sb0, sb1, sw0, sw1):
    c = lax.axis_index("c")
    s = lax.axis_index("s")
    rows = (rows0, rows1)
    slabs = (slab0, slab1)
    build_sems = (sb0, sb1)
    write_sems = (sw0, sw1)
    base = 120 - 8 * s

    def compute_rows(hh, buf):
        h = HEADS_PER_CORE * c + hh
        pltpu.sync_copy(
            diag_hbm.at[
                pl.ds(
                    pl.multiple_of((h * 8 + 7) * D2_LEN + base, 8), WIN_LEN
                )
            ],
            win_v,
        )
        for i in range(8):
            def cbody(cv, carry, i=i):
                rows[buf][i, pl.ds(16 * cv, 16)] = win_v[
                    pl.ds(7 - i + 16 * cv, 16)
                ]
                return carry
            lax.fori_loop(0, SLAB_W // 16, cbody, 0)

    def build(buf):
        return pltpu.async_copy(
            rows[buf],
            slabs[buf].at[pl.ds(pl.multiple_of(8 * s, 8), 8), :],
            build_sems[buf],
        )

    def write(hh, buf):
        h = HEADS_PER_CORE * c + hh
        t = pl.multiple_of(128 * (15 - s), 128)
        return pltpu.async_copy(
            slabs[buf].at[:, pl.ds(t, S_K)],
            out_hbm.at[h, pl.ds(pl.multiple_of(128 * s, 8), 128), :],
            write_sems[buf],
        )

    n = HEADS_PER_CORE
    build_cp = [None] * n
    write_cp = [None] * n
    compute_rows(0, 0)
    build_cp[0] = build(0)
    for hh in range(n):
        buf = hh % 2
        build_cp[hh].wait()
        plsc.subcore_barrier()
        write_cp[hh] = write(hh, buf)
        if hh + 1 < n:
            if hh >= 1:
                write_cp[hh - 1].wait()
            plsc.subcore_barrier()
            compute_rows(hh + 1, 1 - buf)
            build_cp[hh + 1] = build(1 - buf)
    write_cp[n - 2].wait()
    write_cp[n - 1].wait()


def kernel(seq_length, key_length, relative_attention_bias):
    off = (jnp.asarray(key_length, jnp.int32) - S_K) - (
        jnp.asarray(seq_length, jnp.int32) - S_Q
    )
    off = off.reshape((1,))

    diag = pl.pallas_call(
        _diag_kernel,
        grid=(NUM_HEADS,),
        in_specs=[
            pl.BlockSpec(memory_space=pltpu.SMEM),
            pl.BlockSpec(memory_space=pltpu.SMEM),
        ],
        out_specs=pl.BlockSpec((1, 8, D2_LEN), lambda h: (h, 0, 0)),
        out_shape=jax.ShapeDtypeStruct((NUM_HEADS, 8, D2_LEN), jnp.float32),
    )(off, relative_attention_bias)

    mesh = plsc.VectorSubcoreMesh(core_axis_name="c", subcore_axis_name="s")
    expand = functools.partial(
        pl.kernel,
        mesh=mesh,
        out_type=jax.ShapeDtypeStruct((NUM_HEADS, S_Q, S_K), jnp.float32),
        scratch_types=[
            pltpu.VMEM((WIN_LEN,), jnp.float32),
            pltpu.VMEM((8, SLAB_W), jnp.float32),
            pltpu.VMEM((8, SLAB_W), jnp.float32),
            pltpu.VMEM_SHARED((128, SLAB_W), jnp.float32),
            pltpu.VMEM_SHARED((128, SLAB_W), jnp.float32),
            pltpu.SemaphoreType.DMA,
            pltpu.SemaphoreType.DMA,
            pltpu.SemaphoreType.DMA,
            pltpu.SemaphoreType.DMA,
        ],
    )(_sc_expand)
    return expand(diag.reshape(-1))


# speedup vs baseline: 1.0587x; 1.0587x over previous
import functools
import math

import jax
import jax.numpy as jnp
from jax import lax
from jax.experimental import pallas as pl
from jax.experimental.pallas import tpu as pltpu
from jax.experimental.pallas import tpu_sc as plsc

NUM_BUCKETS = 32
NUM_HEADS = 16
MAX_DISTANCE = 128
S_Q = 2048
S_K = 2048
D2_LEN = 4096
SLAB_W = 3968
HEADS_PER_CORE = NUM_HEADS // 2


def _bucket_values(d, table_ref, h):
    nb = NUM_BUCKETS // 2
    base = jnp.where(d > 0, nb, 0).astype(jnp.int32)
    rp = jnp.abs(d)
    max_exact = nb // 2
    is_small = rp < max_exact
    rpf = rp.astype(jnp.float32)
    large = max_exact + (
        jnp.log(rpf / max_exact)
        / math.log(MAX_DISTANCE / max_exact)
        * (nb - max_exact)
    ).astype(jnp.int32)
    large = jnp.minimum(large, nb - 1)
    bucket = base + jnp.where(is_small, rp, large)
    val = jnp.zeros(d.shape, jnp.float32)
    for b in range(NUM_BUCKETS):
        val = jnp.where(bucket == b, table_ref[b, h], val)
    return val


def _diag_kernel(off_ref, table_ref, out_ref):
    h = pl.program_id(0)
    i = jax.lax.broadcasted_iota(jnp.int32, (8, D2_LEN), 0)
    j = jax.lax.broadcasted_iota(jnp.int32, (8, D2_LEN), 1)
    d = j + 7 - i - 2047 + off_ref[0]
    val = _bucket_values(d, table_ref, h)
    for s in range(16):
        out_ref[0, s] = val[:, 120 - 8 * s : 120 - 8 * s + SLAB_W]


def _sc_expand(diag_hbm, out_hbm, slab0, slab1, sb0, sb1, sw0, sw1):
    c = lax.axis_index("c")
    s = lax.axis_index("s")
    slabs = (slab0, slab1)
    build_sems = (sb0, sb1)
    write_sems = (sw0, sw1)

    def build(hh, buf):
        h = HEADS_PER_CORE * c + hh
        return pltpu.async_copy(
            diag_hbm.at[h, s],
            slabs[buf].at[pl.ds(pl.multiple_of(8 * s, 8), 8), :],
            build_sems[buf],
        )

    def write(hh, buf):
        h = HEADS_PER_CORE * c + hh
        t = pl.multiple_of(128 * (15 - s), 128)
        return pltpu.async_copy(
            slabs[buf].at[:, pl.ds(t, S_K)],
            out_hbm.at[h, pl.ds(pl.multiple_of(128 * s, 8), 128), :],
            write_sems[buf],
        )

    n = HEADS_PER_CORE
    build_cp = [None] * n
    write_cp = [None] * n
    build_cp[0] = build(0, 0)
    for hh in range(n):
        buf = hh % 2
        build_cp[hh].wait()
        plsc.subcore_barrier()
        write_cp[hh] = write(hh, buf)
        if hh + 1 < n:
            if hh >= 1:
                write_cp[hh - 1].wait()
            plsc.subcore_barrier()
            build_cp[hh + 1] = build(hh + 1, 1 - buf)
    write_cp[n - 2].wait()
    write_cp[n - 1].wait()


def kernel(seq_length, key_length, relative_attention_bias):
    off = (jnp.asarray(key_length, jnp.int32) - S_K) - (
        jnp.asarray(seq_length, jnp.int32) - S_Q
    )
    off = off.reshape((1,))

    diag = pl.pallas_call(
        _diag_kernel,
        grid=(NUM_HEADS,),
        in_specs=[
            pl.BlockSpec(memory_space=pltpu.SMEM),
            pl.BlockSpec(memory_space=pltpu.SMEM),
        ],
        out_specs=pl.BlockSpec((1, 16, 8, SLAB_W), lambda h: (h, 0, 0, 0)),
        out_shape=jax.ShapeDtypeStruct(
            (NUM_HEADS, 16, 8, SLAB_W), jnp.float32
        ),
    )(off, relative_attention_bias)

    mesh = plsc.VectorSubcoreMesh(core_axis_name="c", subcore_axis_name="s")
    expand = functools.partial(
        pl.kernel,
        mesh=mesh,
        out_type=jax.ShapeDtypeStruct((NUM_HEADS, S_Q, S_K), jnp.float32),
        scratch_types=[
            pltpu.VMEM_SHARED((128, SLAB_W), jnp.float32),
            pltpu.VMEM_SHARED((128, SLAB_W), jnp.float32),
            pltpu.SemaphoreType.DMA,
            pltpu.SemaphoreType.DMA,
            pltpu.SemaphoreType.DMA,
            pltpu.SemaphoreType.DMA,
        ],
    )(_sc_expand)
    return expand(diag)
